# Initial kernel scaffold; baseline (speedup 1.0000x reference)
import jax
import jax.numpy as jnp
from jax.experimental import pallas as pl


def kernel(tgt_emb, click_emb, click_len, exposure_emb, exposure_len, W1, b1, W2, b2, W3, b3):
    raise NotImplementedError("write your pallas kernel here")


# speedup vs baseline: 1.3739x; 1.3739x over previous
import jax
import jax.numpy as jnp
from jax.experimental import pallas as pl
from jax.experimental.pallas import tpu as pltpu

_K = 50
_NITER = 30
_SPAN = 128.0


def _soft_block(tgt, seq, slen):
    Bb, L, D = seq.shape
    seq_r = seq.astype(jnp.bfloat16).astype(jnp.float32)
    tgt_r = tgt.astype(jnp.bfloat16).astype(jnp.float32)
    scores = jnp.sum(seq_r * tgt_r[:, None, :], axis=2)
    pos = jax.lax.broadcasted_iota(jnp.int32, (Bb, L), 1)
    valid = pos >= (L - slen)
    scores = jnp.where(valid, scores, jnp.float32(-1e9))
    rowmax = jnp.max(scores, axis=1, keepdims=True)
    lo = rowmax - jnp.float32(_SPAN)
    hi = rowmax
    kf = jnp.float32(_K)

    def body(_, carry):
        lo, hi = carry
        mid = 0.5 * (lo + hi)
        cnt = jnp.sum((scores >= mid).astype(jnp.float32), axis=1, keepdims=True)
        ge = cnt >= kf
        return jnp.where(ge, mid, lo), jnp.where(ge, hi, mid)

    lo, _ = jax.lax.fori_loop(0, _NITER, body, (lo, hi))
    w = jnp.where(scores >= lo, jnp.exp(scores - rowmax), jnp.float32(0.0))
    w = jnp.where(slen == 0, (pos < _K).astype(jnp.float32), w)
    w = w / jnp.sum(w, axis=1, keepdims=True)
    return jnp.sum(seq * w[:, :, None], axis=1)


def _fused(tgt_ref, click_ref, clen_ref, exp_ref, elen_ref,
           w1_ref, b1_ref, w2_ref, b2_ref, w3_ref, b3_ref, out_ref):
    tgt = tgt_ref[:]
    c = _soft_block(tgt, click_ref[:], clen_ref[:])
    e = _soft_block(tgt, exp_ref[:], elen_ref[:])
    h = jnp.concatenate([c, e], axis=1)
    h = jnp.maximum(jnp.dot(h, w1_ref[:], preferred_element_type=jnp.float32)
                    + b1_ref[:], 0.0)
    h = jnp.maximum(jnp.dot(h, w2_ref[:], preferred_element_type=jnp.float32)
                    + b2_ref[:], 0.0)
    h = jnp.maximum(jnp.dot(h, w3_ref[:], preferred_element_type=jnp.float32)
                    + b3_ref[:], 0.0)
    out_ref[:] = h


def kernel(tgt_emb, click_emb, click_len, exposure_emb, exposure_len,
           W1, b1, W2, b2, W3, b3):
    B, L, D = click_emb.shape
    Bb = 64
    grid = (B // Bb,)
    clen = click_len.reshape(B, 1)
    elen = exposure_len.reshape(B, 1)
    b1r = b1.reshape(1, -1)
    b2r = b2.reshape(1, -1)
    b3r = b3.reshape(1, -1)
    u1, u2, u3 = W1.shape[1], W2.shape[1], W3.shape[1]

    row = lambda i: (i, 0)
    row3 = lambda i: (i, 0, 0)
    rep = lambda i: (0, 0)

    out = pl.pallas_call(
        _fused,
        grid=grid,
        in_specs=[
            pl.BlockSpec((Bb, D), row),
            pl.BlockSpec((Bb, L, D), row3),
            pl.BlockSpec((Bb, 1), row),
            pl.BlockSpec((Bb, L, D), row3),
            pl.BlockSpec((Bb, 1), row),
            pl.BlockSpec((2 * D, u1), rep),
            pl.BlockSpec((1, u1), rep),
            pl.BlockSpec((u1, u2), rep),
            pl.BlockSpec((1, u2), rep),
            pl.BlockSpec((u2, u3), rep),
            pl.BlockSpec((1, u3), rep),
        ],
        out_specs=pl.BlockSpec((Bb, u3), row),
        out_shape=jax.ShapeDtypeStruct((B, u3), jnp.float32),
        compiler_params=pltpu.CompilerParams(
            dimension_semantics=("arbitrary",),
        ),
    )(tgt_emb, click_emb, clen, exposure_emb, elen,
      W1, b1r, W2, b2r, W3, b3r)
    return out[:, None, :]
